# baseline (speedup 1.0000x reference)
import functools

import jax
import jax.numpy as jnp
from jax import lax
from jax.experimental import pallas as pl
from jax.experimental.pallas import tpu as pltpu
from jax.experimental.pallas import tpu_sc as plsc

N = 10000
E = 320000
D = 128

NC = 2
NS = 16
CH = 125
CPT = 80
N_ACC = 10240

_mesh = plsc.VectorSubcoreMesh(core_axis_name="c", subcore_axis_name="s")


def _degrees_body(src_hbm, dst_hbm, od_hbm, id_hbm, src_v, dst_v, ones_v,
                  zero_v, dsem, od_sh, id_sh):
    cid = lax.axis_index("c")
    sid = lax.axis_index("s")
    pltpu.sync_copy(src_hbm.at[cid, sid], src_v)
    pltpu.sync_copy(dst_hbm.at[cid, sid], dst_v)

    def fill(i, _):
        ones_v[pl.ds(i * 16, 16)] = jnp.ones((16,), jnp.float32)
        return 0

    lax.fori_loop(0, 128 // 16, fill, 0)

    def zfill(i, _):
        zero_v[pl.ds(i * 16, 16)] = jnp.zeros((16,), jnp.float32)
        return 0

    lax.fori_loop(0, 640 // 16, zfill, 0)

    pltpu.sync_copy(zero_v, od_sh.at[pl.ds(sid * 640, 640)])
    pltpu.sync_copy(zero_v, id_sh.at[pl.ds(sid * 640, 640)])
    plsc.subcore_barrier()

    ones_c = ones_v.at[pl.ds(0, CH)]

    def body(j, _):
        pltpu.async_copy(ones_c, od_sh.at[src_v.at[j]], dsem, add=True)
        pltpu.async_copy(ones_c, id_sh.at[dst_v.at[j]], dsem, add=True)
        return 0

    lax.fori_loop(0, CPT, body, 0)

    def drain(j, _):
        pltpu.make_async_copy(ones_c, od_sh.at[src_v.at[j]], dsem).wait()
        pltpu.make_async_copy(ones_c, id_sh.at[dst_v.at[j]], dsem).wait()
        return 0

    lax.fori_loop(0, CPT, drain, 0)
    plsc.subcore_barrier()

    off = cid * N_ACC + sid * 640
    pltpu.sync_copy(od_sh.at[pl.ds(sid * 640, 640)],
                    od_hbm.at[pl.ds(off, 640)])
    pltpu.sync_copy(id_sh.at[pl.ds(sid * 640, 640)],
                    id_hbm.at[pl.ds(off, 640)])


HCPT = CPT // 2


def _aggregate_body(h_hbm, src_hbm, dst_hbm, out_hbm, src_v, dst_v,
                    buf0, buf1, sem0, sem1, ssem0, ssem1, acc_sh):
    cid = lax.axis_index("c")
    sid = lax.axis_index("s")

    def zrow(i, _):
        for k in range(D // 16):
            buf0[i, pl.ds(k * 16, 16)] = jnp.zeros((16,), jnp.float32)
        return 0

    lax.fori_loop(0, CH, zrow, 0)
    trows = sid * (N_ACC // NS)
    for k in range(5):
        pltpu.sync_copy(buf0, acc_sh.at[pl.ds(trows + k * CH, CH)])
    pltpu.sync_copy(buf0.at[pl.ds(0, 15)],
                    acc_sh.at[pl.ds(trows + 5 * CH, 15)])
    plsc.subcore_barrier()

    def body(i, _):
        j0 = 2 * i
        j1 = 2 * i + 1
        pltpu.make_async_copy(h_hbm.at[src_v.at[j0]], buf0, sem0).wait()
        pltpu.async_copy(buf0, acc_sh.at[dst_v.at[j0]], ssem0, add=True)
        pltpu.make_async_copy(h_hbm.at[src_v.at[j1]], buf1, sem1).wait()
        pltpu.async_copy(buf1, acc_sh.at[dst_v.at[j1]], ssem1, add=True)

        pltpu.make_async_copy(buf0, acc_sh.at[dst_v.at[j0]], ssem0).wait()

        @pl.when(j0 + 2 < HCPT)
        def _():
            pltpu.async_copy(h_hbm.at[src_v.at[j0 + 2]], buf0, sem0)

        pltpu.make_async_copy(buf1, acc_sh.at[dst_v.at[j1]], ssem1).wait()

        @pl.when(j1 + 2 < HCPT)
        def _():
            pltpu.async_copy(h_hbm.at[src_v.at[j1 + 2]], buf1, sem1)

        return 0

    for half in range(2):
        pltpu.sync_copy(src_hbm.at[cid, sid, pl.ds(half * HCPT, HCPT)], src_v)
        pltpu.sync_copy(dst_hbm.at[cid, sid, pl.ds(half * HCPT, HCPT)], dst_v)
        pltpu.async_copy(h_hbm.at[src_v.at[0]], buf0, sem0)
        pltpu.async_copy(h_hbm.at[src_v.at[1]], buf1, sem1)
        lax.fori_loop(0, HCPT // 2, body, 0)
    plsc.subcore_barrier()

    rows = N_ACC // NS
    pltpu.sync_copy(acc_sh.at[pl.ds(sid * rows, rows)],
                    out_hbm.at[cid, pl.ds(sid * rows, rows)])


BN = 400


def _linear_tc_body(deg_ref, x_ref, w_ref, o_ref):
    dp = deg_ref[...]
    s = lax.rsqrt(jnp.maximum(dp[:, 0] + dp[:, 1], 1.0))
    o_ref[...] = jnp.dot(x_ref[...] * s[:, None], w_ref[...],
                         preferred_element_type=jnp.float32)


def _final_tc_body(p_ref, deg_ref, b_ref, o_ref):
    a = p_ref[0] + p_ref[1]
    dp = deg_ref[...]
    s = lax.rsqrt(jnp.maximum(dp[:, 0] + dp[:, 1], 1.0))
    o_ref[...] = a * s[:, None] + b_ref[...]


def _make_degrees():
    @functools.partial(
        pl.kernel,
        out_type=(
            jax.ShapeDtypeStruct((NC * N_ACC,), jnp.float32),
            jax.ShapeDtypeStruct((NC * N_ACC,), jnp.float32),
        ),
        mesh=_mesh,
        scratch_types=[
            pltpu.VMEM((CPT, CH), jnp.int32),
            pltpu.VMEM((CPT, CH), jnp.int32),
            pltpu.VMEM((128,), jnp.float32),
            pltpu.VMEM((640,), jnp.float32),
            pltpu.SemaphoreType.DMA,
            pltpu.VMEM_SHARED((N_ACC,), jnp.float32),
            pltpu.VMEM_SHARED((N_ACC,), jnp.float32),
        ],
    )
    def deg_kernel(src_hbm, dst_hbm, od_hbm, id_hbm, src_v, dst_v, ones_v,
                   zero_v, dsem, od_sh, id_sh):
        _degrees_body(src_hbm, dst_hbm, od_hbm, id_hbm, src_v, dst_v, ones_v,
                      zero_v, dsem, od_sh, id_sh)

    return deg_kernel


def _make_aggregate():
    @functools.partial(
        pl.kernel,
        out_type=jax.ShapeDtypeStruct((NC, N_ACC, D), jnp.float32),
        mesh=_mesh,
        scratch_types=[
            pltpu.VMEM((HCPT, CH), jnp.int32),
            pltpu.VMEM((HCPT, CH), jnp.int32),
            pltpu.VMEM((CH, D), jnp.float32),
            pltpu.VMEM((CH, D), jnp.float32),
            pltpu.SemaphoreType.DMA,
            pltpu.SemaphoreType.DMA,
            pltpu.SemaphoreType.DMA,
            pltpu.SemaphoreType.DMA,
            pltpu.VMEM_SHARED((N_ACC, D), jnp.float32),
        ],
    )
    def agg_kernel(h_hbm, src_hbm, dst_hbm, out_hbm, src_v, dst_v,
                   buf0, buf1, sem0, sem1, ssem0, ssem1, acc_sh):
        _aggregate_body(h_hbm, src_hbm, dst_hbm, out_hbm, src_v, dst_v,
                        buf0, buf1, sem0, sem1, ssem0, ssem1, acc_sh)

    return agg_kernel


_deg_kernel = _make_degrees()
_agg_kernel = _make_aggregate()

_linear_tc = pl.pallas_call(
    _linear_tc_body,
    grid=(N // BN,),
    in_specs=[
        pl.BlockSpec((BN, NC), lambda i: (i, 0)),
        pl.BlockSpec((BN, D), lambda i: (i, 0)),
        pl.BlockSpec((D, D), lambda i: (0, 0)),
    ],
    out_specs=pl.BlockSpec((BN, D), lambda i: (i, 0)),
    out_shape=jax.ShapeDtypeStruct((N, D), jnp.float32),
)

_final_tc = pl.pallas_call(
    _final_tc_body,
    grid=(N // BN,),
    in_specs=[
        pl.BlockSpec((NC, BN, D), lambda i: (0, i, 0)),
        pl.BlockSpec((BN, NC), lambda i: (i, 0)),
        pl.BlockSpec((D,), lambda i: (0,)),
    ],
    out_specs=pl.BlockSpec((BN, D), lambda i: (i, 0)),
    out_shape=jax.ShapeDtypeStruct((N, D), jnp.float32),
)


def kernel(x, edge_index, W, b):
    src = edge_index[0].reshape(NC, NS, CPT, CH)
    dst = edge_index[1].reshape(NC, NS, CPT, CH)

    od_p, id_p = _deg_kernel(src, dst)
    h = _linear_tc(od_p.reshape(NC, N_ACC).T, x, W)
    parts = _agg_kernel(h, src, dst)
    return _final_tc(parts, id_p.reshape(NC, N_ACC).T, b)


# speedup vs baseline: 1.3735x; 1.3735x over previous
import functools

import jax
import jax.numpy as jnp
from jax import lax
from jax.experimental import pallas as pl
from jax.experimental.pallas import tpu as pltpu
from jax.experimental.pallas import tpu_sc as plsc

N = 10000
E = 320000
D = 128

NC = 2
NS = 16
CH = 125
CPT = 80
N_ACC = 10240

_mesh = plsc.VectorSubcoreMesh(core_axis_name="c", subcore_axis_name="s")


def _degrees_body(src_hbm, dst_hbm, so_hbm, si_hbm, idx_v, ones_v,
                  zero_v, sbuf, dsem, deg_sh):
    cid = lax.axis_index("c")
    sid = lax.axis_index("s")

    def fill(i, _):
        ones_v[pl.ds(i * 16, 16)] = jnp.ones((16,), jnp.float32)
        return 0

    lax.fori_loop(0, 128 // 16, fill, 0)

    def zfill(i, _):
        zero_v[pl.ds(i * 16, 16)] = jnp.zeros((16,), jnp.float32)
        return 0

    lax.fori_loop(0, 640 // 16, zfill, 0)

    pltpu.sync_copy(zero_v, deg_sh.at[pl.ds(sid * 640, 640)])
    plsc.subcore_barrier()

    ones_c = ones_v.at[pl.ds(0, CH)]

    def body(j, _):
        pltpu.async_copy(ones_c, deg_sh.at[idx_v.at[j]], dsem, add=True)
        return 0

    def drain(j, _):
        pltpu.make_async_copy(ones_c, deg_sh.at[idx_v.at[j]], dsem).wait()
        return 0

    for part in range(NC):
        @pl.when(cid == 0)
        def _():
            pltpu.sync_copy(src_hbm.at[part, sid], idx_v)

        @pl.when(cid == 1)
        def _():
            pltpu.sync_copy(dst_hbm.at[part, sid], idx_v)

        lax.fori_loop(0, CPT, body, 0)
        lax.fori_loop(0, CPT, drain, 0)
    plsc.subcore_barrier()

    pltpu.sync_copy(deg_sh.at[pl.ds(sid * 640, 640)], sbuf)

    def nr(i, _):
        d = jnp.maximum(sbuf[pl.ds(i * 16, 16)], 1.0)
        bits = lax.bitcast_convert_type(d, jnp.int32)
        y = lax.bitcast_convert_type(
            jnp.int32(0x5F3759DF) - lax.shift_right_logical(bits, 1),
            jnp.float32)
        for _ in range(3):
            y = y * (1.5 - 0.5 * d * y * y)
        sbuf[pl.ds(i * 16, 16)] = y
        return 0

    lax.fori_loop(0, 640 // 16, nr, 0)

    @pl.when(cid == 0)
    def _():
        pltpu.sync_copy(sbuf, so_hbm.at[pl.ds(sid * 640, 640)])

    @pl.when(cid == 1)
    def _():
        pltpu.sync_copy(sbuf, si_hbm.at[pl.ds(sid * 640, 640)])


HCPT = CPT // 2


def _aggregate_body(h_hbm, src_hbm, dst_hbm, out_hbm, src_v, dst_v,
                    buf0, buf1, sem0, sem1, acc_sh):
    cid = lax.axis_index("c")
    sid = lax.axis_index("s")

    def zrow(i, _):
        for k in range(D // 16):
            buf0[i, pl.ds(k * 16, 16)] = jnp.zeros((16,), jnp.float32)
        return 0

    lax.fori_loop(0, CH, zrow, 0)
    trows = sid * (N_ACC // NS)
    for k in range(5):
        pltpu.sync_copy(buf0, acc_sh.at[pl.ds(trows + k * CH, CH)])
    pltpu.sync_copy(buf0.at[pl.ds(0, 15)],
                    acc_sh.at[pl.ds(trows + 5 * CH, 15)])
    plsc.subcore_barrier()

    def body(i, _):
        j0 = 2 * i
        j1 = 2 * i + 1
        pltpu.make_async_copy(h_hbm.at[src_v.at[j0]], buf0, sem0).wait()
        pltpu.sync_copy(buf0, acc_sh.at[dst_v.at[j0]], add=True)

        @pl.when(j0 + 2 < HCPT)
        def _():
            pltpu.async_copy(h_hbm.at[src_v.at[j0 + 2]], buf0, sem0)

        pltpu.make_async_copy(h_hbm.at[src_v.at[j1]], buf1, sem1).wait()
        pltpu.sync_copy(buf1, acc_sh.at[dst_v.at[j1]], add=True)

        @pl.when(j1 + 2 < HCPT)
        def _():
            pltpu.async_copy(h_hbm.at[src_v.at[j1 + 2]], buf1, sem1)

        return 0

    for half in range(2):
        pltpu.sync_copy(src_hbm.at[cid, sid, pl.ds(half * HCPT, HCPT)], src_v)
        pltpu.sync_copy(dst_hbm.at[cid, sid, pl.ds(half * HCPT, HCPT)], dst_v)
        pltpu.async_copy(h_hbm.at[src_v.at[0]], buf0, sem0)
        pltpu.async_copy(h_hbm.at[src_v.at[1]], buf1, sem1)
        lax.fori_loop(0, HCPT // 2, body, 0)
    plsc.subcore_barrier()

    rows = N_ACC // NS
    pltpu.sync_copy(acc_sh.at[pl.ds(sid * rows, rows)],
                    out_hbm.at[cid, pl.ds(sid * rows, rows)])


BN = 2048


def _linear_tc_body(s_ref, x_ref, w_ref, o_ref):
    s = s_ref[...]
    o_ref[...] = jnp.dot(x_ref[...] * s[:, None], w_ref[...],
                         preferred_element_type=jnp.float32)


def _final_tc_body(p_ref, s_ref, b_ref, o_ref):
    a = p_ref[0] + p_ref[1]
    s = s_ref[...]
    o_ref[...] = a * s[:, None] + b_ref[...]


def _make_degrees():
    @functools.partial(
        pl.kernel,
        out_type=(
            jax.ShapeDtypeStruct((N_ACC,), jnp.float32),
            jax.ShapeDtypeStruct((N_ACC,), jnp.float32),
        ),
        mesh=_mesh,
        scratch_types=[
            pltpu.VMEM((CPT, CH), jnp.int32),
            pltpu.VMEM((128,), jnp.float32),
            pltpu.VMEM((640,), jnp.float32),
            pltpu.VMEM((640,), jnp.float32),
            pltpu.SemaphoreType.DMA,
            pltpu.VMEM_SHARED((N_ACC,), jnp.float32),
        ],
    )
    def deg_kernel(src_hbm, dst_hbm, so_hbm, si_hbm, idx_v, ones_v,
                   zero_v, sbuf, dsem, deg_sh):
        _degrees_body(src_hbm, dst_hbm, so_hbm, si_hbm, idx_v, ones_v,
                      zero_v, sbuf, dsem, deg_sh)

    return deg_kernel


def _make_aggregate():
    @functools.partial(
        pl.kernel,
        out_type=jax.ShapeDtypeStruct((NC, N_ACC, D), jnp.float32),
        mesh=_mesh,
        scratch_types=[
            pltpu.VMEM((HCPT, CH), jnp.int32),
            pltpu.VMEM((HCPT, CH), jnp.int32),
            pltpu.VMEM((CH, D), jnp.float32),
            pltpu.VMEM((CH, D), jnp.float32),
            pltpu.SemaphoreType.DMA,
            pltpu.SemaphoreType.DMA,
            pltpu.VMEM_SHARED((N_ACC, D), jnp.float32),
        ],
    )
    def agg_kernel(h_hbm, src_hbm, dst_hbm, out_hbm, src_v, dst_v,
                   buf0, buf1, sem0, sem1, acc_sh):
        _aggregate_body(h_hbm, src_hbm, dst_hbm, out_hbm, src_v, dst_v,
                        buf0, buf1, sem0, sem1, acc_sh)

    return agg_kernel


_deg_kernel = _make_degrees()
_agg_kernel = _make_aggregate()

_GRID = N_ACC // BN

_linear_tc = pl.pallas_call(
    _linear_tc_body,
    grid=(_GRID,),
    in_specs=[
        pl.BlockSpec((BN,), lambda i: (i,)),
        pl.BlockSpec((BN, D), lambda i: (i, 0)),
        pl.BlockSpec((D, D), lambda i: (0, 0)),
    ],
    out_specs=pl.BlockSpec((BN, D), lambda i: (i, 0)),
    out_shape=jax.ShapeDtypeStruct((N, D), jnp.float32),
)

_final_tc = pl.pallas_call(
    _final_tc_body,
    grid=(_GRID,),
    in_specs=[
        pl.BlockSpec((NC, BN, D), lambda i: (0, i, 0)),
        pl.BlockSpec((BN,), lambda i: (i,)),
        pl.BlockSpec((D,), lambda i: (0,)),
    ],
    out_specs=pl.BlockSpec((BN, D), lambda i: (i, 0)),
    out_shape=jax.ShapeDtypeStruct((N, D), jnp.float32),
)


def kernel(x, edge_index, W, b):
    src = edge_index[0].reshape(NC, NS, CPT, CH)
    dst = edge_index[1].reshape(NC, NS, CPT, CH)

    s_out, s_in = _deg_kernel(src, dst)
    h = _linear_tc(s_out, x, W)
    parts = _agg_kernel(h, src, dst)
    return _final_tc(parts, s_in, b)
